# Initial kernel scaffold; baseline (speedup 1.0000x reference)
import math

import jax
import jax.numpy as jnp
from jax.experimental import pallas as pl

EMBED = 64
HALF = 32
NTYPES = 100
TPAD = 128

B, S = 4096, 200
N = B * S
BLK_ROWS = 64
BLK = BLK_ROWS * 128
GRID = N // BLK


def _tc_body(dts_ref, types_ref, table_ref, tw_ref, tb_ref, ew_ref, eb_ref, out_ref):
    ftab = (table_ref[...] * math.sqrt(EMBED // 2)) @ ew_ref[HALF:, :]
    c = tb_ref[...] @ ew_ref[:HALF, :] + eb_ref[...]
    v = tw_ref[...] @ ew_ref[:HALF, :]
    ftab_c = ftab + c

    dt = jnp.log(dts_ref[...] + 1e-08)
    dt_flat = dt.reshape(BLK, 1)
    types = types_ref[...].reshape(BLK, 1)
    onehot = (types == jax.lax.broadcasted_iota(jnp.int32, (BLK, TPAD), 1))
    z = jnp.dot(onehot.astype(jnp.float32), ftab_c,
                preferred_element_type=jnp.float32)
    z = z + dt_flat * v
    out_ref[...] = jnp.tanh(z)


def kernel(seq_dts, seq_types, type_table, time_W, time_b, enc_W, enc_b):
    types2 = seq_types.astype(jnp.int32).reshape(N // 128, 128)
    dts2 = seq_dts.reshape(N // 128, 128)
    table_pad = jnp.pad(type_table, ((0, TPAD - (NTYPES + 1)), (0, 0)))
    tb2 = time_b.reshape(1, HALF)
    eb2 = enc_b.reshape(1, EMBED)

    out = pl.pallas_call(
        _tc_body,
        grid=(GRID,),
        in_specs=[
            pl.BlockSpec((BLK_ROWS, 128), lambda i: (i, 0)),
            pl.BlockSpec((BLK_ROWS, 128), lambda i: (i, 0)),
            pl.BlockSpec((TPAD, HALF), lambda i: (0, 0)),
            pl.BlockSpec((1, HALF), lambda i: (0, 0)),
            pl.BlockSpec((1, HALF), lambda i: (0, 0)),
            pl.BlockSpec((EMBED, EMBED), lambda i: (0, 0)),
            pl.BlockSpec((1, EMBED), lambda i: (0, 0)),
        ],
        out_specs=pl.BlockSpec((BLK, EMBED), lambda i: (i, 0)),
        out_shape=jax.ShapeDtypeStruct((N, EMBED), jnp.float32),
    )(dts2, types2, table_pad, time_W, tb2, enc_W, eb2)
    return out.reshape(B, S, EMBED)


# speedup vs baseline: 8.0506x; 8.0506x over previous
import math

import jax
import jax.numpy as jnp
from jax.experimental import pallas as pl

EMBED = 64
HALF = 32
NTYPES = 100
TPAD = 128

B, S = 4096, 200
N = B * S
BLK = 8192
GRID = N // BLK


def _tc_body(dts_ref, types_ref, table_ref, tw_ref, tb_ref, ew_ref, eb_ref, out_ref):
    ftab = (table_ref[...] * math.sqrt(EMBED // 2)) @ ew_ref[HALF:, :]
    c = tb_ref[...] @ ew_ref[:HALF, :] + eb_ref[...]
    v = tw_ref[...] @ ew_ref[:HALF, :]
    row = jax.lax.broadcasted_iota(jnp.int32, (TPAD, EMBED), 0)
    ftab_full = jnp.where(row == TPAD - 1, v, ftab + c)

    types = types_ref[0]
    dt = jnp.log(dts_ref[0] + 1e-08)
    tid = jax.lax.broadcasted_iota(jnp.int32, (TPAD, BLK), 0)
    onehot_t = (tid == types).astype(jnp.float32)
    lhs = jnp.where(tid == TPAD - 1, dt, onehot_t)

    z = jax.lax.dot_general(lhs, ftab_full,
                            dimension_numbers=(((0,), (0,)), ((), ())),
                            preferred_element_type=jnp.float32)
    out_ref[...] = jnp.tanh(z)


def kernel(seq_dts, seq_types, type_table, time_W, time_b, enc_W, enc_b):
    types3 = seq_types.astype(jnp.int32).reshape(GRID, 1, BLK)
    dts3 = seq_dts.reshape(GRID, 1, BLK)
    table_pad = jnp.pad(type_table, ((0, TPAD - (NTYPES + 1)), (0, 0)))
    tb2 = time_b.reshape(1, HALF)
    eb2 = enc_b.reshape(1, EMBED)

    out = pl.pallas_call(
        _tc_body,
        grid=(GRID,),
        in_specs=[
            pl.BlockSpec((1, 1, BLK), lambda i: (i, 0, 0)),
            pl.BlockSpec((1, 1, BLK), lambda i: (i, 0, 0)),
            pl.BlockSpec((TPAD, HALF), lambda i: (0, 0)),
            pl.BlockSpec((1, HALF), lambda i: (0, 0)),
            pl.BlockSpec((1, HALF), lambda i: (0, 0)),
            pl.BlockSpec((EMBED, EMBED), lambda i: (0, 0)),
            pl.BlockSpec((1, EMBED), lambda i: (0, 0)),
        ],
        out_specs=pl.BlockSpec((BLK, EMBED), lambda i: (i, 0)),
        out_shape=jax.ShapeDtypeStruct((N, EMBED), jnp.float32),
    )(dts3, types3, table_pad, time_W, tb2, enc_W, eb2)
    return out.reshape(B, S, EMBED)
